# baseline (speedup 1.0000x reference)
import functools

import jax
import jax.numpy as jnp
from jax import lax
from jax.experimental import pallas as pl
from jax.experimental.pallas import tpu as pltpu
from jax.experimental.pallas import tpu_sc as plsc

_NC = 2
_NS = 16
_NW = _NC * _NS


def _fc1_body(x_ref, w_ref, b_ref, o_ref):
    o_ref[...] = (
        lax.dot_general(x_ref[...], w_ref[...], (((1,), (1,)), ((), ())),
                        preferred_element_type=jnp.float32)
        + b_ref[...]
    )


def _fc1(x, w, b):
    n, d = x.shape
    blk = 5000
    return pl.pallas_call(
        _fc1_body,
        grid=(n // blk,),
        in_specs=[
            pl.BlockSpec((blk, d), lambda i: (i, 0)),
            pl.BlockSpec((d, d), lambda i: (0, 0)),
            pl.BlockSpec((1, d), lambda i: (0, 0)),
        ],
        out_specs=pl.BlockSpec((blk, d), lambda i: (i, 0)),
        out_shape=jax.ShapeDtypeStruct((n, d), jnp.float32),
    )(x, w, b.reshape(1, d))


def _make_seg_sum(n, d, e):
    k = 128
    nblk = e // k
    nit = nblk // _NW
    nextra = nblk - nit * _NW
    assert nit % 3 == 0
    rows_pt = (n // _NS) // 8 * 8
    rtail = n - rows_pt * _NS

    mesh = plsc.VectorSubcoreMesh(core_axis_name="c", subcore_axis_name="s")

    @functools.partial(
        pl.kernel,
        out_type=jax.ShapeDtypeStruct((_NC, n, d), jnp.float32),
        mesh=mesh,
        scratch_types=[
            pltpu.VMEM((2, k), jnp.int32),
            pltpu.VMEM((2, k), jnp.int32),
            pltpu.VMEM((2, k), jnp.int32),
            pltpu.VMEM((k, d), jnp.float32),
            pltpu.VMEM((k, d), jnp.float32),
            pltpu.VMEM((k, d), jnp.float32),
            pltpu.VMEM_SHARED((n, d), jnp.float32),
            pltpu.SemaphoreType.DMA,
            pltpu.SemaphoreType.DMA,
            pltpu.SemaphoreType.DMA,
            pltpu.SemaphoreType.DMA,
            pltpu.SemaphoreType.DMA,
            pltpu.SemaphoreType.DMA,
        ],
    )
    def seg_sum(h_hbm, ei3_hbm, out_hbm,
                idx0, idx1, idx2, rows0, rows1, rows2, acc_sh,
                sg0, sg1, sg2, si0, si1, si2):
        cid = lax.axis_index("c")
        sid = lax.axis_index("s")
        wid = sid * _NC + cid

        def zrow(r, carry):
            for c in range(d // 16):
                rows0[r, pl.ds(c * 16, 16)] = jnp.zeros((16,), jnp.float32)
            return carry

        lax.fori_loop(0, k, zrow, 0)
        for i in range(rows_pt // k):
            pltpu.sync_copy(rows0,
                            acc_sh.at[pl.ds(sid * rows_pt + i * k, k)])
        zrem = rows_pt % k
        if zrem:
            pltpu.sync_copy(
                rows0.at[pl.ds(0, zrem)],
                acc_sh.at[pl.ds(sid * rows_pt + (rows_pt // k) * k, zrem)])
        if rtail:
            @pl.when(sid == _NS - 1)
            def _():
                pltpu.sync_copy(rows0.at[pl.ds(0, rtail)],
                                acc_sh.at[pl.ds(rows_pt * _NS, rtail)])
        plsc.subcore_barrier()

        idx = [idx0, idx1, idx2]
        rows = [rows0, rows1, rows2]
        sg = [sg0, sg1, sg2]
        si = [si0, si1, si2]
        blk0 = wid * nit

        def iload(c, s):
            pltpu.async_copy(ei3_hbm.at[blk0 + c], idx[s], si[s])

        def iwait(s):
            pltpu.make_async_copy(ei3_hbm.at[0], idx[s], si[s]).wait()

        def gather(s):
            pltpu.async_copy(h_hbm.at[idx[s].at[0]], rows[s], sg[s])

        def gwait(s):
            pltpu.make_async_copy(h_hbm.at[pl.ds(0, k)], rows[s],
                                  sg[s]).wait()

        def scatter(s):
            pltpu.sync_copy(rows[s], acc_sh.at[idx[s].at[1]], add=True)

        iload(0, 0)
        iload(1, 1)
        iload(2, 2)
        iwait(0)
        gather(0)
        iwait(1)
        gather(1)

        def body(j, carry):
            c0 = j * 3
            for s in range(3):
                c = c0 + s
                gwait(s)
                scatter(s)

                @pl.when(c + 3 < nit)
                def _():
                    iload(c + 3, s)

                @pl.when(c + 2 < nit)
                def _():
                    iwait((s + 2) % 3)
                    gather((s + 2) % 3)

            return carry

        lax.fori_loop(0, nit // 3, body, 0)

        if nextra:
            @pl.when(wid < nextra)
            def _():
                pltpu.async_copy(ei3_hbm.at[_NW * nit + wid], idx0, si0)
                iwait(0)
                gather(0)
                gwait(0)
                scatter(0)

        plsc.subcore_barrier()
        pltpu.sync_copy(acc_sh.at[pl.ds(sid * rows_pt, rows_pt)],
                        out_hbm.at[cid, pl.ds(sid * rows_pt, rows_pt)])
        if rtail:
            @pl.when(sid == _NS - 1)
            def _():
                pltpu.sync_copy(acc_sh.at[pl.ds(rows_pt * _NS, rtail)],
                                out_hbm.at[cid, pl.ds(rows_pt * _NS, rtail)])

    return seg_sum


def _mlp_body(scale_ref, h1_ref, a0_ref, a1_ref,
              w1_ref, b1_ref, w2_ref, b2_ref, bng_ref, bnb_ref,
              l1w_ref, l1b_ref, l2w_ref, l2b_ref, o_ref):
    u = scale_ref[0, 0] * h1_ref[...] + a0_ref[0] + a1_ref[0]
    dn = (((1,), (1,)), ((), ()))
    t = jnp.maximum(
        lax.dot_general(u, w1_ref[...], dn, preferred_element_type=jnp.float32)
        + b1_ref[...], 0.0)
    t = jnp.maximum(
        lax.dot_general(t, w2_ref[...], dn, preferred_element_type=jnp.float32)
        + b2_ref[...], 0.0)
    t = t * (bng_ref[...] * (1.0 / jnp.sqrt(1.0 + 1e-5))) + bnb_ref[...]
    t = jnp.maximum(
        lax.dot_general(t, l1w_ref[...], dn, preferred_element_type=jnp.float32)
        + l1b_ref[...], 0.0)
    o_ref[...] = (
        lax.dot_general(t, l2w_ref[...], dn, preferred_element_type=jnp.float32)
        + l2b_ref[...])


def _mlp(scale, h1, aggs, w1, b1, w2, b2, bn_g, bn_b, l1w, l1b, l2w, l2b):
    n, d = h1.shape
    h = w1.shape[0]
    blk = 2000
    full = lambda shape: pl.BlockSpec(shape, lambda i: tuple(0 for _ in shape))
    row = lambda width: pl.BlockSpec((blk, width), lambda i: (i, 0))
    return pl.pallas_call(
        _mlp_body,
        grid=(n // blk,),
        in_specs=[
            pl.BlockSpec(memory_space=pltpu.SMEM),
            row(d),
            pl.BlockSpec((1, blk, d), lambda i: (0, i, 0)),
            pl.BlockSpec((1, blk, d), lambda i: (1, i, 0)),
            full((h, d)), full((1, h)), full((h, h)), full((1, h)),
            full((1, h)), full((1, h)),
            full((h, h)), full((1, h)), full((h, h)), full((1, h)),
        ],
        out_specs=row(h),
        out_shape=jax.ShapeDtypeStruct((n, h), jnp.float32),
    )(scale, h1, aggs, aggs, w1, b1.reshape(1, h), w2, b2.reshape(1, h),
      bn_g.reshape(1, h), bn_b.reshape(1, h),
      l1w, l1b.reshape(1, h), l2w, l2b.reshape(1, h))


def _make_pair_gather(n, h, e, b):
    bpt = b // _NW
    m = 64
    nsub = bpt // m
    assert bpt % m == 0 and nsub % 2 == 0
    nv = h // 16

    mesh = plsc.VectorSubcoreMesh(core_axis_name="c", subcore_axis_name="s")

    @functools.partial(
        pl.kernel,
        out_type=jax.ShapeDtypeStruct((b, h), jnp.float32),
        mesh=mesh,
        scratch_types=[
            pltpu.VMEM((bpt,), jnp.int32),
            pltpu.VMEM((bpt,), jnp.int32),
            pltpu.VMEM((bpt,), jnp.int32),
            pltpu.VMEM((nsub, m), jnp.int32),
            pltpu.VMEM((nsub, m), jnp.int32),
            pltpu.VMEM((m, h), jnp.float32),
            pltpu.VMEM((m, h), jnp.float32),
            pltpu.VMEM((m, h), jnp.float32),
            pltpu.VMEM((m, h), jnp.float32),
            pltpu.SemaphoreType.DMA,
            pltpu.SemaphoreType.DMA,
            pltpu.SemaphoreType.DMA,
        ],
    )
    def pair(h_hbm, flat_hbm, te_hbm, out_hbm,
             te_v, t0_v, t1_v, s_v, d_v, x1a, x2a, x1b, x2b,
             sem1, sem2, sem_i):
        cid = lax.axis_index("c")
        sid = lax.axis_index("s")
        wid = sid * _NC + cid
        base = wid * bpt
        pltpu.sync_copy(te_hbm.at[pl.ds(base, bpt)], te_v)

        def tmap(i, carry):
            sl = pl.ds(i * 16, 16)
            t = te_v[sl]
            a = ((t >> 7) << 8) | (t & 127)
            t0_v[sl] = a
            t1_v[sl] = a + 128
            return carry

        lax.fori_loop(0, bpt // 16, tmap, 0)

        for j in range(nsub):
            pltpu.async_copy(flat_hbm.at[t0_v.at[pl.ds(j * m, m)]],
                             s_v.at[j], sem_i)
            pltpu.async_copy(flat_hbm.at[t1_v.at[pl.ds(j * m, m)]],
                             d_v.at[j], sem_i)
        for j in range(2 * nsub):
            pltpu.make_async_copy(flat_hbm.at[pl.ds(0, m)],
                                  s_v.at[0], sem_i).wait()

        def xgather(j, x1, x2):
            pltpu.async_copy(h_hbm.at[s_v.at[j]], x1, sem1)
            pltpu.async_copy(h_hbm.at[d_v.at[j]], x2, sem2)

        def xwait(x1, x2):
            pltpu.make_async_copy(h_hbm.at[pl.ds(0, m)], x1, sem1).wait()
            pltpu.make_async_copy(h_hbm.at[pl.ds(0, m)], x2, sem2).wait()

        def mul_write(j, x1, x2):
            def mul_row(r2, carry):
                for u in range(2):
                    r = r2 * 2 + u
                    for c in range(nv):
                        sl = pl.ds(c * 16, 16)
                        x1[r, sl] = x1[r, sl] * x2[r, sl]
                return carry

            lax.fori_loop(0, m // 2, mul_row, 0)
            pltpu.sync_copy(x1, out_hbm.at[pl.ds(base + j * m, m)])

        xgather(0, x1a, x2a)
        for j in range(0, nsub, 2):
            xwait(x1a, x2a)
            if j + 1 < nsub:
                xgather(j + 1, x1b, x2b)
            mul_write(j, x1a, x2a)
            xwait(x1b, x2b)
            if j + 2 < nsub:
                xgather(j + 2, x1a, x2a)
            mul_write(j + 1, x1b, x2b)

    return pair


def _fc2_body(f_ref, w_ref, b_ref, o_ref):
    o_ref[...] = (
        lax.dot_general(w_ref[...], f_ref[...], (((1,), (1,)), ((), ())),
                        preferred_element_type=jnp.float32)
        + b_ref[...])


def _fc2(fused, w, bias):
    b, h = fused.shape
    c = w.shape[0]
    blk = 4096
    out_t = pl.pallas_call(
        _fc2_body,
        grid=(b // blk,),
        in_specs=[
            pl.BlockSpec((blk, h), lambda i: (i, 0)),
            pl.BlockSpec((c, h), lambda i: (0, 0)),
            pl.BlockSpec((c, 1), lambda i: (0, 0)),
        ],
        out_specs=pl.BlockSpec((c, blk), lambda i: (0, i)),
        out_shape=jax.ShapeDtypeStruct((c, b), jnp.float32),
    )(fused, w, bias.reshape(c, 1))
    return out_t.T


def kernel(x, edge_index, train_edge_id, fc1_w, fc1_b, eps, w1, b1, w2, b2,
           bn_g, bn_b, lin1_w, lin1_b, lin2_w, lin2_b, fc2_w, fc2_b):
    n, d = x.shape
    e = edge_index.shape[1]
    b = train_edge_id.shape[0]
    h = w1.shape[0]

    ei3 = edge_index.reshape(2, e // 128, 128).swapaxes(0, 1)
    flat = ei3.reshape(2 * e)

    h1 = _fc1(x, fc1_w, fc1_b)

    aggs = _make_seg_sum(n, d, e)(h1, ei3)

    scale = (1.0 + eps).reshape(1, 1)
    h5 = _mlp(scale, h1, aggs, w1, b1, w2, b2, bn_g, bn_b,
              lin1_w, lin1_b, lin2_w, lin2_b)

    fused = _make_pair_gather(n, h, e, b)(h5, flat, train_edge_id)

    return _fc2(fused, fc2_w, fc2_b)


# speedup vs baseline: 1.0075x; 1.0075x over previous
import functools

import jax
import jax.numpy as jnp
from jax import lax
from jax.experimental import pallas as pl
from jax.experimental.pallas import tpu as pltpu
from jax.experimental.pallas import tpu_sc as plsc

_NC = 2
_NS = 16
_NW = _NC * _NS


def _fc1_body(x_ref, w_ref, b_ref, o_ref):
    o_ref[...] = (
        lax.dot_general(x_ref[...], w_ref[...], (((1,), (1,)), ((), ())),
                        preferred_element_type=jnp.float32)
        + b_ref[...]
    )


def _fc1(x, w, b):
    n, d = x.shape
    blk = 5000
    return pl.pallas_call(
        _fc1_body,
        grid=(n // blk,),
        in_specs=[
            pl.BlockSpec((blk, d), lambda i: (i, 0)),
            pl.BlockSpec((d, d), lambda i: (0, 0)),
            pl.BlockSpec((1, d), lambda i: (0, 0)),
        ],
        out_specs=pl.BlockSpec((blk, d), lambda i: (i, 0)),
        out_shape=jax.ShapeDtypeStruct((n, d), jnp.float32),
    )(x, w, b.reshape(1, d))


def _make_seg_sum(n, d, e):
    k = 128
    nblk = e // k
    nit = nblk // _NW
    nextra = nblk - nit * _NW
    assert nit % 3 == 0
    rows_pt = (n // _NS) // 8 * 8
    rtail = n - rows_pt * _NS

    mesh = plsc.VectorSubcoreMesh(core_axis_name="c", subcore_axis_name="s")

    @functools.partial(
        pl.kernel,
        out_type=jax.ShapeDtypeStruct((_NC, n, d), jnp.float32),
        mesh=mesh,
        scratch_types=[
            pltpu.VMEM((2, k), jnp.int32),
            pltpu.VMEM((2, k), jnp.int32),
            pltpu.VMEM((2, k), jnp.int32),
            pltpu.VMEM((k, d), jnp.float32),
            pltpu.VMEM((k, d), jnp.float32),
            pltpu.VMEM((k, d), jnp.float32),
            pltpu.VMEM_SHARED((n, d), jnp.float32),
            pltpu.SemaphoreType.DMA,
            pltpu.SemaphoreType.DMA,
            pltpu.SemaphoreType.DMA,
            pltpu.SemaphoreType.DMA,
            pltpu.SemaphoreType.DMA,
            pltpu.SemaphoreType.DMA,
        ],
    )
    def seg_sum(h_hbm, ei3_hbm, out_hbm,
                idx0, idx1, idx2, rows0, rows1, rows2, acc_sh,
                sg0, sg1, sg2, si0, si1, si2):
        cid = lax.axis_index("c")
        sid = lax.axis_index("s")
        wid = sid * _NC + cid

        def zrow(r, carry):
            for c in range(d // 16):
                rows0[r, pl.ds(c * 16, 16)] = jnp.zeros((16,), jnp.float32)
            return carry

        lax.fori_loop(0, k, zrow, 0)
        for i in range(rows_pt // k):
            pltpu.sync_copy(rows0,
                            acc_sh.at[pl.ds(sid * rows_pt + i * k, k)])
        zrem = rows_pt % k
        if zrem:
            pltpu.sync_copy(
                rows0.at[pl.ds(0, zrem)],
                acc_sh.at[pl.ds(sid * rows_pt + (rows_pt // k) * k, zrem)])
        if rtail:
            @pl.when(sid == _NS - 1)
            def _():
                pltpu.sync_copy(rows0.at[pl.ds(0, rtail)],
                                acc_sh.at[pl.ds(rows_pt * _NS, rtail)])
        plsc.subcore_barrier()

        idx = [idx0, idx1, idx2]
        rows = [rows0, rows1, rows2]
        sg = [sg0, sg1, sg2]
        si = [si0, si1, si2]
        blk0 = wid * nit

        def iload(c, s):
            pltpu.async_copy(ei3_hbm.at[blk0 + c], idx[s], si[s])

        def iwait(s):
            pltpu.make_async_copy(ei3_hbm.at[0], idx[s], si[s]).wait()

        def gather(s):
            pltpu.async_copy(h_hbm.at[idx[s].at[0]], rows[s], sg[s])

        def gwait(s):
            pltpu.make_async_copy(h_hbm.at[pl.ds(0, k)], rows[s],
                                  sg[s]).wait()

        def scatter(s):
            pltpu.sync_copy(rows[s], acc_sh.at[idx[s].at[1]], add=True)

        iload(0, 0)
        iload(1, 1)
        iload(2, 2)
        iwait(0)
        gather(0)
        iwait(1)
        gather(1)

        def body(j, carry):
            c0 = j * 3
            for s in range(3):
                c = c0 + s
                gwait(s)
                scatter(s)

                @pl.when(c + 3 < nit)
                def _():
                    iload(c + 3, s)

                @pl.when(c + 2 < nit)
                def _():
                    iwait((s + 2) % 3)
                    gather((s + 2) % 3)

            return carry

        lax.fori_loop(0, nit // 3, body, 0)

        if nextra:
            @pl.when(wid < nextra)
            def _():
                pltpu.async_copy(ei3_hbm.at[_NW * nit + wid], idx0, si0)
                iwait(0)
                gather(0)
                gwait(0)
                scatter(0)

        plsc.subcore_barrier()
        pltpu.sync_copy(acc_sh.at[pl.ds(sid * rows_pt, rows_pt)],
                        out_hbm.at[cid, pl.ds(sid * rows_pt, rows_pt)])
        if rtail:
            @pl.when(sid == _NS - 1)
            def _():
                pltpu.sync_copy(acc_sh.at[pl.ds(rows_pt * _NS, rtail)],
                                out_hbm.at[cid, pl.ds(rows_pt * _NS, rtail)])

    return seg_sum


def _mlp_body(scale_ref, h1_ref, a0_ref, a1_ref,
              w1_ref, b1_ref, w2_ref, b2_ref, bng_ref, bnb_ref,
              l1w_ref, l1b_ref, l2w_ref, l2b_ref, o_ref):
    u = scale_ref[0, 0] * h1_ref[...] + a0_ref[0] + a1_ref[0]
    dn = (((1,), (1,)), ((), ()))
    t = jnp.maximum(
        lax.dot_general(u, w1_ref[...], dn, preferred_element_type=jnp.float32)
        + b1_ref[...], 0.0)
    t = jnp.maximum(
        lax.dot_general(t, w2_ref[...], dn, preferred_element_type=jnp.float32)
        + b2_ref[...], 0.0)
    t = t * (bng_ref[...] * (1.0 / jnp.sqrt(1.0 + 1e-5))) + bnb_ref[...]
    t = jnp.maximum(
        lax.dot_general(t, l1w_ref[...], dn, preferred_element_type=jnp.float32)
        + l1b_ref[...], 0.0)
    o_ref[...] = (
        lax.dot_general(t, l2w_ref[...], dn, preferred_element_type=jnp.float32)
        + l2b_ref[...])


def _mlp(scale, h1, aggs, w1, b1, w2, b2, bn_g, bn_b, l1w, l1b, l2w, l2b):
    n, d = h1.shape
    h = w1.shape[0]
    blk = 2000
    full = lambda shape: pl.BlockSpec(shape, lambda i: tuple(0 for _ in shape))
    row = lambda width: pl.BlockSpec((blk, width), lambda i: (i, 0))
    return pl.pallas_call(
        _mlp_body,
        grid=(n // blk,),
        in_specs=[
            pl.BlockSpec(memory_space=pltpu.SMEM),
            row(d),
            pl.BlockSpec((1, blk, d), lambda i: (0, i, 0)),
            pl.BlockSpec((1, blk, d), lambda i: (1, i, 0)),
            full((h, d)), full((1, h)), full((h, h)), full((1, h)),
            full((1, h)), full((1, h)),
            full((h, h)), full((1, h)), full((h, h)), full((1, h)),
        ],
        out_specs=row(h),
        out_shape=jax.ShapeDtypeStruct((n, h), jnp.float32),
    )(scale, h1, aggs, aggs, w1, b1.reshape(1, h), w2, b2.reshape(1, h),
      bn_g.reshape(1, h), bn_b.reshape(1, h),
      l1w, l1b.reshape(1, h), l2w, l2b.reshape(1, h))


def _make_pair_gather(n, h, e, b):
    bpt = b // _NW
    m = 64
    nsub = bpt // m
    assert bpt % m == 0 and nsub % 2 == 0
    nv = h // 16

    mesh = plsc.VectorSubcoreMesh(core_axis_name="c", subcore_axis_name="s")

    @functools.partial(
        pl.kernel,
        out_type=jax.ShapeDtypeStruct((b, h), jnp.float32),
        mesh=mesh,
        scratch_types=[
            pltpu.VMEM((bpt,), jnp.int32),
            pltpu.VMEM((bpt,), jnp.int32),
            pltpu.VMEM((bpt,), jnp.int32),
            pltpu.VMEM((nsub, m), jnp.int32),
            pltpu.VMEM((nsub, m), jnp.int32),
            pltpu.VMEM((m, h), jnp.float32),
            pltpu.VMEM((m, h), jnp.float32),
            pltpu.VMEM((m, h), jnp.float32),
            pltpu.VMEM((m, h), jnp.float32),
            pltpu.SemaphoreType.DMA,
            pltpu.SemaphoreType.DMA,
            pltpu.SemaphoreType.DMA,
        ],
    )
    def pair(h_hbm, flat_hbm, te_hbm, out_hbm,
             te_v, t0_v, t1_v, s_v, d_v, x1a, x2a, x1b, x2b,
             sem1, sem2, sem_i):
        cid = lax.axis_index("c")
        sid = lax.axis_index("s")
        wid = sid * _NC + cid
        base = wid * bpt
        pltpu.sync_copy(te_hbm.at[pl.ds(base, bpt)], te_v)

        def tmap(i, carry):
            sl = pl.ds(i * 16, 16)
            t = te_v[sl]
            a = ((t >> 7) << 8) | (t & 127)
            t0_v[sl] = a
            t1_v[sl] = a + 128
            return carry

        lax.fori_loop(0, bpt // 16, tmap, 0)

        for j in range(nsub):
            pltpu.async_copy(flat_hbm.at[t0_v.at[pl.ds(j * m, m)]],
                             s_v.at[j], sem_i)
            pltpu.async_copy(flat_hbm.at[t1_v.at[pl.ds(j * m, m)]],
                             d_v.at[j], sem_i)
        for j in range(2 * nsub):
            pltpu.make_async_copy(flat_hbm.at[pl.ds(0, m)],
                                  s_v.at[0], sem_i).wait()

        def xgather(j, x1, x2):
            pltpu.async_copy(h_hbm.at[s_v.at[j]], x1, sem1)
            pltpu.async_copy(h_hbm.at[d_v.at[j]], x2, sem2)

        def xwait(x1, x2):
            pltpu.make_async_copy(h_hbm.at[pl.ds(0, m)], x1, sem1).wait()
            pltpu.make_async_copy(h_hbm.at[pl.ds(0, m)], x2, sem2).wait()

        def mul_write(j, x1, x2):
            def mul_row(r, carry):
                for c in range(nv):
                    sl = pl.ds(c * 16, 16)
                    x1[r, sl] = x1[r, sl] * x2[r, sl]
                return carry

            lax.fori_loop(0, m, mul_row, 0)
            pltpu.sync_copy(x1, out_hbm.at[pl.ds(base + j * m, m)])

        xgather(0, x1a, x2a)
        for j in range(0, nsub, 2):
            xwait(x1a, x2a)
            if j + 1 < nsub:
                xgather(j + 1, x1b, x2b)
            mul_write(j, x1a, x2a)
            xwait(x1b, x2b)
            if j + 2 < nsub:
                xgather(j + 2, x1a, x2a)
            mul_write(j + 1, x1b, x2b)

    return pair


def _fc2_body(f_ref, w_ref, b_ref, o_ref):
    o_ref[...] = (
        lax.dot_general(w_ref[...], f_ref[...], (((1,), (1,)), ((), ())),
                        preferred_element_type=jnp.float32)
        + b_ref[...])


def _fc2(fused, w, bias):
    b, h = fused.shape
    c = w.shape[0]
    blk = 4096
    out_t = pl.pallas_call(
        _fc2_body,
        grid=(b // blk,),
        in_specs=[
            pl.BlockSpec((blk, h), lambda i: (i, 0)),
            pl.BlockSpec((c, h), lambda i: (0, 0)),
            pl.BlockSpec((c, 1), lambda i: (0, 0)),
        ],
        out_specs=pl.BlockSpec((c, blk), lambda i: (0, i)),
        out_shape=jax.ShapeDtypeStruct((c, b), jnp.float32),
    )(fused, w, bias.reshape(c, 1))
    return out_t.T


def kernel(x, edge_index, train_edge_id, fc1_w, fc1_b, eps, w1, b1, w2, b2,
           bn_g, bn_b, lin1_w, lin1_b, lin2_w, lin2_b, fc2_w, fc2_b):
    n, d = x.shape
    e = edge_index.shape[1]
    b = train_edge_id.shape[0]
    h = w1.shape[0]

    ei3 = edge_index.reshape(2, e // 128, 128).swapaxes(0, 1)
    flat = ei3.reshape(2 * e)

    h1 = _fc1(x, fc1_w, fc1_b)

    aggs = _make_seg_sum(n, d, e)(h1, ei3)

    scale = (1.0 + eps).reshape(1, 1)
    h5 = _mlp(scale, h1, aggs, w1, b1, w2, b2, bn_g, bn_b,
              lin1_w, lin1_b, lin2_w, lin2_b)

    fused = _make_pair_gather(n, h, e, b)(h5, flat, train_edge_id)

    return _fc2(fused, fc2_w, fc2_b)
